# baseline (speedup 1.0000x reference)
import jax
import jax.numpy as jnp
from jax import lax
from jax.experimental import pallas as pl
from jax.experimental.pallas import tpu as pltpu
from jax.experimental.pallas import tpu_sc as plsc

N = 10000
E = 160000
D = 256
HALF = 128
NC = 2
NS = 16
K = 96
NRB = 3
NIB = 6
G = 2
F = 3
EPT = E // NS
W = ((EPT + K - 1) // K + NIB - 1) // NIB * NIB
PADE = W * K - EPT
NPAD = 16
NA = N + NPAD
ROWS_PER_TILE = 624
ROWS_TAIL = N - NS * ROWS_PER_TILE


def _sc_agg_body(x_hbm, e4_hbm, out_hbm, idx_v, rows_v, acc,
                 sem_i, sem_g, sem_s):
    c = lax.axis_index("c")
    s = lax.axis_index("s")

    table = x_hbm.at[:, pl.ds(c * HALF, HALF)]

    base_r = s * ROWS_PER_TILE
    pltpu.sync_copy(table.at[pl.ds(base_r, ROWS_PER_TILE)],
                    acc.at[pl.ds(base_r, ROWS_PER_TILE)])

    @pl.when(s == NS - 1)
    def _():
        pltpu.sync_copy(
            table.at[pl.ds(NS * ROWS_PER_TILE, ROWS_TAIL)],
            acc.at[pl.ds(NS * ROWS_PER_TILE, ROWS_TAIL)])

    plsc.subcore_barrier()

    e4 = e4_hbm.at[s]

    def fire_idx(j, si):
        return pltpu.async_copy(e4.at[j], idx_v.at[si], sem_i.at[si])

    def wait_idx(j, si):
        pltpu.make_async_copy(e4.at[j], idx_v.at[si], sem_i.at[si]).wait()

    KH = K // 2

    def fire_gather(b, si):
        pltpu.async_copy(table.at[idx_v.at[si, 0, pl.ds(0, KH)]],
                         rows_v.at[b, pl.ds(0, KH)], sem_g.at[b])
        pltpu.async_copy(table.at[idx_v.at[si, 0, pl.ds(KH, KH)]],
                         rows_v.at[b, pl.ds(KH, KH)], sem_g.at[b])

    def wait_gather(b, si):
        pltpu.make_async_copy(table.at[idx_v.at[si, 0, pl.ds(0, KH)]],
                              rows_v.at[b, pl.ds(0, KH)], sem_g.at[b]).wait()
        pltpu.make_async_copy(table.at[idx_v.at[si, 0, pl.ds(KH, KH)]],
                              rows_v.at[b, pl.ds(KH, KH)], sem_g.at[b]).wait()

    def fire_scatter(b, si):
        return pltpu.async_copy(rows_v.at[b], acc.at[idx_v.at[si, 1]],
                                sem_s.at[b], add=True)

    def wait_scatter(b, si):
        pltpu.make_async_copy(rows_v.at[b], acc.at[idx_v.at[si, 1]],
                              sem_s.at[b]).wait()

    def step(j, o, prologue):
        b = o % NRB
        if not prologue or o >= NRB:
            wait_scatter(b, (o - NRB) % NIB)
        wait_idx(j, o)
        fire_gather(b, o)
        if not prologue or o >= G:
            wait_gather((o - G) % NRB, (o - G) % NIB)
            fire_scatter((o - G) % NRB, (o - G) % NIB)
        if prologue:
            fire_idx(j + F, (o + F) % NIB)
        else:
            @pl.when(j + F < W)
            def _():
                fire_idx(j + F, (o + F) % NIB)

    for jf in range(F):
        fire_idx(jf, jf)
    for o in range(NIB):
        step(o, o, True)

    def body(r, carry):
        jb = r * NIB
        for o in range(NIB):
            step(jb + o, o, False)
        return carry

    lax.fori_loop(1, W // NIB, body, 0)

    for w in range(W - G, W):
        wait_gather(w % NRB, w % NIB)
        fire_scatter(w % NRB, w % NIB)
    for w in range(W - NRB, W):
        wait_scatter(w % NRB, w % NIB)

    plsc.subcore_barrier()

    pltpu.sync_copy(acc.at[pl.ds(base_r, ROWS_PER_TILE)],
                    out_hbm.at[c].at[pl.ds(base_r, ROWS_PER_TILE)])

    @pl.when(s == NS - 1)
    def _():
        pltpu.sync_copy(
            acc.at[pl.ds(NS * ROWS_PER_TILE, ROWS_TAIL)],
            out_hbm.at[c].at[pl.ds(NS * ROWS_PER_TILE, ROWS_TAIL)])


_sc_agg = pl.kernel(
    _sc_agg_body,
    out_type=jax.ShapeDtypeStruct((NC, N, HALF), jnp.float32),
    mesh=plsc.VectorSubcoreMesh(core_axis_name="c", subcore_axis_name="s"),
    scratch_types=[
        pltpu.VMEM((NIB, 2, K), jnp.int32),
        pltpu.VMEM((NRB, K, HALF), jnp.float32),
        pltpu.VMEM_SHARED((NA, HALF), jnp.float32),
        pltpu.SemaphoreType.DMA((NIB,)),
        pltpu.SemaphoreType.DMA((NRB,)),
        pltpu.SemaphoreType.DMA((NRB,)),
    ],
)


def _tc_weight_body(mem_ref, wih_ref, bi_ref, bh_ref, wt_ref, bt_ref,
                    out_ref):
    xt = mem_ref[...]
    gi_r = jnp.dot(wih_ref[0], xt, preferred_element_type=jnp.float32)
    gi_z = jnp.dot(wih_ref[1], xt, preferred_element_type=jnp.float32)
    gi_n = jnp.dot(wih_ref[2], xt, preferred_element_type=jnp.float32)
    r = jax.nn.sigmoid(gi_r + bi_ref[0] + bh_ref[0])
    z = jax.nn.sigmoid(gi_z + bi_ref[1] + bh_ref[1])
    n = jnp.tanh(gi_n + bi_ref[2] + r * bh_ref[2])
    u = (1.0 - z) * n
    nw = jnp.dot(wt_ref[...], u, preferred_element_type=jnp.float32)
    out_ref[...] = nw + bt_ref[...]


def _tc_out_body(h_ref, wl_ref, b_ref, out_ref):
    dn = (((1,), (1,)), ((), ()))
    acc = jax.lax.dot_general(h_ref[0], wl_ref[:, 0:HALF], dn,
                              preferred_element_type=jnp.float32)
    acc = acc + jax.lax.dot_general(h_ref[1], wl_ref[:, HALF:D], dn,
                                    preferred_element_type=jnp.float32)
    out_ref[...] = jnp.maximum(acc + b_ref[...], 0.0)


def kernel(x, edge_index, memory, W_ih, W_hh, b_ih, b_hh, W_t, b_t, b_lin):
    del W_hh

    pad_src = (jnp.arange(NS, dtype=jnp.int32)[:, None] * 617
               + jnp.arange(PADE, dtype=jnp.int32) * 7) % N
    pad_dst = jnp.broadcast_to(
        N + jnp.arange(PADE, dtype=jnp.int32) % NPAD, (NS, PADE))
    src4 = jnp.concatenate(
        [edge_index[0].reshape(NS, EPT), pad_src], axis=1).reshape(NS, W, K)
    dst4 = jnp.concatenate(
        [edge_index[1].reshape(NS, EPT), pad_dst], axis=1).reshape(NS, W, K)
    e4 = jnp.stack([src4, dst4], axis=2)

    wih3 = W_ih.reshape(3, D, D)
    bi = b_ih.reshape(3, D, 1)
    bh = b_hh.reshape(3, D, 1)
    bt = b_t.reshape(D * D, 1)
    memT = memory.reshape(D, 1)

    h2 = _sc_agg(x, e4)

    wt_blk = 8192
    n_wt = (D * D) // wt_blk
    new_w = pl.pallas_call(
        _tc_weight_body,
        grid=(n_wt,),
        in_specs=[
            pl.BlockSpec((D, 1), lambda k: (0, 0)),
            pl.BlockSpec((3, D, D), lambda k: (0, 0, 0)),
            pl.BlockSpec((3, D, 1), lambda k: (0, 0, 0)),
            pl.BlockSpec((3, D, 1), lambda k: (0, 0, 0)),
            pl.BlockSpec((wt_blk, D), lambda k: (k, 0)),
            pl.BlockSpec((wt_blk, 1), lambda k: (k, 0)),
        ],
        out_specs=pl.BlockSpec((wt_blk, 1), lambda k: (k, 0)),
        out_shape=jax.ShapeDtypeStruct((D * D, 1), jnp.float32),
    )(memT, wih3, bi, bh, W_t, bt)

    w_lin = new_w.reshape(D, D)

    row_blk = 1000
    out = pl.pallas_call(
        _tc_out_body,
        grid=(N // row_blk,),
        in_specs=[
            pl.BlockSpec((NC, row_blk, HALF), lambda i: (0, i, 0)),
            pl.BlockSpec((D, D), lambda i: (0, 0)),
            pl.BlockSpec((1, D), lambda i: (0, 0)),
        ],
        out_specs=pl.BlockSpec((row_blk, D), lambda i: (i, 0)),
        out_shape=jax.ShapeDtypeStruct((N, D), jnp.float32),
    )(h2, w_lin, b_lin.reshape(1, D))
    return out


# speedup vs baseline: 1.0705x; 1.0705x over previous
import jax
import jax.numpy as jnp
from jax import lax
from jax.experimental import pallas as pl
from jax.experimental.pallas import tpu as pltpu
from jax.experimental.pallas import tpu_sc as plsc

N = 10000
E = 160000
D = 256
HALF = 128
NC = 2
NS = 16
K = 80
NRB = 3
NIB = 6
G = 2
F = 3
EPT = E // NS
W = ((EPT + K - 1) // K + NIB - 1) // NIB * NIB
PADE = W * K - EPT
NPAD = 16
NA = N + NPAD
ROWS_PER_TILE = 624
ROWS_TAIL = N - NS * ROWS_PER_TILE


def _sc_agg_body(x_hbm, e4_hbm, out_hbm, idx_v, rows_v, acc,
                 sem_i, sem_g, sem_s):
    c = lax.axis_index("c")
    s = lax.axis_index("s")

    table = x_hbm.at[:, pl.ds(c * HALF, HALF)]

    base_r = s * ROWS_PER_TILE
    pltpu.sync_copy(table.at[pl.ds(base_r, ROWS_PER_TILE)],
                    acc.at[pl.ds(base_r, ROWS_PER_TILE)])

    @pl.when(s == NS - 1)
    def _():
        pltpu.sync_copy(
            table.at[pl.ds(NS * ROWS_PER_TILE, ROWS_TAIL)],
            acc.at[pl.ds(NS * ROWS_PER_TILE, ROWS_TAIL)])

    plsc.subcore_barrier()

    e4 = e4_hbm.at[s]

    def fire_idx(j, si):
        return pltpu.async_copy(e4.at[j], idx_v.at[si], sem_i.at[si])

    def wait_idx(j, si):
        pltpu.make_async_copy(e4.at[j], idx_v.at[si], sem_i.at[si]).wait()

    KH = K // 2

    def fire_gather(b, si):
        pltpu.async_copy(table.at[idx_v.at[si, 0, pl.ds(0, KH)]],
                         rows_v.at[b, pl.ds(0, KH)], sem_g.at[b])
        pltpu.async_copy(table.at[idx_v.at[si, 0, pl.ds(KH, KH)]],
                         rows_v.at[b, pl.ds(KH, KH)], sem_g.at[b])

    def wait_gather(b, si):
        pltpu.make_async_copy(table.at[idx_v.at[si, 0, pl.ds(0, KH)]],
                              rows_v.at[b, pl.ds(0, KH)], sem_g.at[b]).wait()
        pltpu.make_async_copy(table.at[idx_v.at[si, 0, pl.ds(KH, KH)]],
                              rows_v.at[b, pl.ds(KH, KH)], sem_g.at[b]).wait()

    def fire_scatter(b, si):
        return pltpu.async_copy(rows_v.at[b], acc.at[idx_v.at[si, 1]],
                                sem_s.at[b], add=True)

    def wait_scatter(b, si):
        pltpu.make_async_copy(rows_v.at[b], acc.at[idx_v.at[si, 1]],
                              sem_s.at[b]).wait()

    def step(j, o, prologue):
        b = o % NRB
        if not prologue or o >= NRB:
            wait_scatter(b, (o - NRB) % NIB)
        wait_idx(j, o)
        fire_gather(b, o)
        if not prologue or o >= G:
            wait_gather((o - G) % NRB, (o - G) % NIB)
            fire_scatter((o - G) % NRB, (o - G) % NIB)
        if prologue:
            fire_idx(j + F, (o + F) % NIB)
        else:
            @pl.when(j + F < W)
            def _():
                fire_idx(j + F, (o + F) % NIB)

    for jf in range(F):
        fire_idx(jf, jf)
    for o in range(NIB):
        step(o, o, True)

    def body(r, carry):
        jb = r * NIB
        for o in range(NIB):
            step(jb + o, o, False)
        return carry

    lax.fori_loop(1, W // NIB, body, 0)

    for w in range(W - G, W):
        wait_gather(w % NRB, w % NIB)
        fire_scatter(w % NRB, w % NIB)
    for w in range(W - NRB, W):
        wait_scatter(w % NRB, w % NIB)

    plsc.subcore_barrier()

    pltpu.sync_copy(acc.at[pl.ds(base_r, ROWS_PER_TILE)],
                    out_hbm.at[c].at[pl.ds(base_r, ROWS_PER_TILE)])

    @pl.when(s == NS - 1)
    def _():
        pltpu.sync_copy(
            acc.at[pl.ds(NS * ROWS_PER_TILE, ROWS_TAIL)],
            out_hbm.at[c].at[pl.ds(NS * ROWS_PER_TILE, ROWS_TAIL)])


_sc_agg = pl.kernel(
    _sc_agg_body,
    out_type=jax.ShapeDtypeStruct((NC, N, HALF), jnp.float32),
    mesh=plsc.VectorSubcoreMesh(core_axis_name="c", subcore_axis_name="s"),
    scratch_types=[
        pltpu.VMEM((NIB, 2, K), jnp.int32),
        pltpu.VMEM((NRB, K, HALF), jnp.float32),
        pltpu.VMEM_SHARED((NA, HALF), jnp.float32),
        pltpu.SemaphoreType.DMA((NIB,)),
        pltpu.SemaphoreType.DMA((NRB,)),
        pltpu.SemaphoreType.DMA((NRB,)),
    ],
)


def _tc_weight_body(mem_ref, wih_ref, bi_ref, bh_ref, wt_ref, bt_ref,
                    out_ref):
    xt = mem_ref[...]
    gi_r = jnp.dot(wih_ref[0], xt, preferred_element_type=jnp.float32)
    gi_z = jnp.dot(wih_ref[1], xt, preferred_element_type=jnp.float32)
    gi_n = jnp.dot(wih_ref[2], xt, preferred_element_type=jnp.float32)
    r = jax.nn.sigmoid(gi_r + bi_ref[0] + bh_ref[0])
    z = jax.nn.sigmoid(gi_z + bi_ref[1] + bh_ref[1])
    n = jnp.tanh(gi_n + bi_ref[2] + r * bh_ref[2])
    u = (1.0 - z) * n
    nw = jnp.dot(wt_ref[...], u, preferred_element_type=jnp.float32)
    out_ref[...] = nw + bt_ref[...]


def _tc_out_body(h_ref, wl_ref, b_ref, out_ref):
    dn = (((1,), (1,)), ((), ()))
    acc = jax.lax.dot_general(h_ref[0], wl_ref[:, 0:HALF], dn,
                              preferred_element_type=jnp.float32)
    acc = acc + jax.lax.dot_general(h_ref[1], wl_ref[:, HALF:D], dn,
                                    preferred_element_type=jnp.float32)
    out_ref[...] = jnp.maximum(acc + b_ref[...], 0.0)


def kernel(x, edge_index, memory, W_ih, W_hh, b_ih, b_hh, W_t, b_t, b_lin):
    del W_hh

    WR = EPT // K
    e4r = edge_index.reshape(2, NS, WR, K).transpose(1, 2, 0, 3)
    pad_src = (jnp.arange(NS, dtype=jnp.int32)[:, None, None, None] * 617
               + jnp.arange((W - WR) * K, dtype=jnp.int32)
               .reshape(1, W - WR, 1, K) * 7) % N
    pad_dst = jnp.broadcast_to(
        N + jnp.arange(K, dtype=jnp.int32) % NPAD, (NS, W - WR, 1, K))
    e4 = jnp.concatenate(
        [e4r, jnp.concatenate([pad_src, pad_dst], axis=2)], axis=1)

    wih3 = W_ih.reshape(3, D, D)
    bi = b_ih.reshape(3, D, 1)
    bh = b_hh.reshape(3, D, 1)
    bt = b_t.reshape(D * D, 1)
    memT = memory.reshape(D, 1)

    h2 = _sc_agg(x, e4)

    wt_blk = 8192
    n_wt = (D * D) // wt_blk
    new_w = pl.pallas_call(
        _tc_weight_body,
        grid=(n_wt,),
        in_specs=[
            pl.BlockSpec((D, 1), lambda k: (0, 0)),
            pl.BlockSpec((3, D, D), lambda k: (0, 0, 0)),
            pl.BlockSpec((3, D, 1), lambda k: (0, 0, 0)),
            pl.BlockSpec((3, D, 1), lambda k: (0, 0, 0)),
            pl.BlockSpec((wt_blk, D), lambda k: (k, 0)),
            pl.BlockSpec((wt_blk, 1), lambda k: (k, 0)),
        ],
        out_specs=pl.BlockSpec((wt_blk, 1), lambda k: (k, 0)),
        out_shape=jax.ShapeDtypeStruct((D * D, 1), jnp.float32),
    )(memT, wih3, bi, bh, W_t, bt)

    w_lin = new_w.reshape(D, D)

    row_blk = 1000
    out = pl.pallas_call(
        _tc_out_body,
        grid=(N // row_blk,),
        in_specs=[
            pl.BlockSpec((NC, row_blk, HALF), lambda i: (0, i, 0)),
            pl.BlockSpec((D, D), lambda i: (0, 0)),
            pl.BlockSpec((1, D), lambda i: (0, 0)),
        ],
        out_specs=pl.BlockSpec((row_blk, D), lambda i: (i, 0)),
        out_shape=jax.ShapeDtypeStruct((N, D), jnp.float32),
    )(h2, w_lin, b_lin.reshape(1, D))
    return out
